# Initial kernel scaffold; baseline (speedup 1.0000x reference)
import jax
import jax.numpy as jnp
from jax.experimental import pallas as pl


def kernel(latents, W):
    raise NotImplementedError("write your pallas kernel here")


# speedup vs baseline: 7.2975x; 7.2975x over previous
import functools

import jax
import jax.numpy as jnp
from jax import lax
from jax.experimental import pallas as pl
from jax.experimental.pallas import tpu as pltpu
from jax.experimental.pallas import tpu_sc as plsc

_D = 256
_K = 8192
_BETA = 0.25

_M = 512
_KB = 1024


def _dist_argmin_body(rn_ref, wn_ref, flat_ref, w_ref, idx_ref, loss_ref,
                      rmin_ref, rarg_ref, *, kb, prec=None):
    j = pl.program_id(1)
    nj = pl.num_programs(1)
    flat_b = flat_ref[...]
    w_b = w_ref[...]
    dot = lax.dot_general(flat_b, w_b, (((1,), (1,)), ((), ())),
                          preferred_element_type=jnp.float32,
                          precision=prec)
    dist = (rn_ref[...] + wn_ref[...]) - 2.0 * dot
    bmin = jnp.min(dist, axis=1, keepdims=True)
    cols = lax.broadcasted_iota(jnp.int32, dist.shape, 1) + j * kb
    barg = jnp.min(jnp.where(dist == bmin, cols, jnp.int32(2**30)),
                   axis=1, keepdims=True)

    @pl.when(j == 0)
    def _init():
        rmin_ref[...] = bmin
        rarg_ref[...] = barg

    @pl.when(j > 0)
    def _update():
        take = bmin < rmin_ref[...]
        rmin_ref[...] = jnp.where(take, bmin, rmin_ref[...])
        rarg_ref[...] = jnp.where(take, barg, rarg_ref[...])

    @pl.when(j == nj - 1)
    def _finish():
        idx_ref[...] = rarg_ref[...]
        i = pl.program_id(0)
        part = jnp.sum(rmin_ref[...]).reshape(1, 1)
        prev = jnp.where(i == 0, jnp.zeros((1, 1), jnp.float32), loss_ref[...])
        loss_ref[...] = prev + part


def _dist_argmin(rn, wn_row, flat, w, prec=None):
    n = flat.shape[0]
    grid = (n // _M, _K // _KB)
    return pl.pallas_call(
        functools.partial(_dist_argmin_body, kb=_KB, prec=prec),
        grid=grid,
        in_specs=[
            pl.BlockSpec((_M, 1), lambda i, j: (i, 0)),
            pl.BlockSpec((1, _KB), lambda i, j: (0, j)),
            pl.BlockSpec((_M, _D), lambda i, j: (i, 0)),
            pl.BlockSpec((_KB, _D), lambda i, j: (j, 0)),
        ],
        out_specs=[
            pl.BlockSpec((_M, 1), lambda i, j: (i, 0)),
            pl.BlockSpec((1, 1), lambda i, j: (0, 0)),
        ],
        out_shape=[
            jax.ShapeDtypeStruct((n, 1), jnp.int32),
            jax.ShapeDtypeStruct((1, 1), jnp.float32),
        ],
        scratch_shapes=[
            pltpu.VMEM((_M, 1), jnp.float32),
            pltpu.VMEM((_M, 1), jnp.int32),
        ],
        compiler_params=pltpu.CompilerParams(
            dimension_semantics=("arbitrary", "arbitrary")),
    )(rn, wn_row, flat, w)


def _sc_gather(w, idx):
    b = idx.shape[0]
    info = plsc.get_sparse_core_info()
    nw = info.num_cores * info.num_subcores
    b_per_w = b // nw
    chunk = 128
    nchunk = b_per_w // chunk
    mesh = plsc.VectorSubcoreMesh(core_axis_name="c", subcore_axis_name="s")

    @functools.partial(
        pl.kernel, mesh=mesh,
        out_type=jax.ShapeDtypeStruct((b, _D), jnp.float32),
        scratch_types=[
            pltpu.VMEM((chunk,), jnp.int32),
            pltpu.VMEM((chunk, _D), jnp.float32),
            pltpu.SemaphoreType.DMA,
        ],
    )
    def gather_kernel(w_hbm, idx_hbm, out_hbm, idx_v, rows_v, sem):
        wid = lax.axis_index("s") * info.num_cores + lax.axis_index("c")
        base = wid * b_per_w
        for c in range(nchunk):
            off = base + c * chunk
            pltpu.sync_copy(idx_hbm.at[pl.ds(off, chunk)], idx_v)
            pltpu.async_copy(w_hbm.at[idx_v], rows_v, sem).wait()
            pltpu.sync_copy(rows_v, out_hbm.at[pl.ds(off, chunk)])

    return gather_kernel(w, idx)


def kernel(latents, W):
    flat = latents.reshape(-1, _D)
    rn = jnp.sum(flat ** 2, axis=1, keepdims=True)
    wn = jnp.sum(W ** 2, axis=1)
    idx, loss_sum = _dist_argmin(rn, wn.reshape(1, _K), flat, W)
    q = _sc_gather(W, idx.reshape(-1)).reshape(latents.shape)
    quantized_st = latents + (q - latents)
    n_elems = flat.shape[0] * _D
    vq_loss = loss_sum[0, 0] * ((1.0 + _BETA) / n_elems)
    return (quantized_st, vq_loss)
